# Initial kernel scaffold; baseline (speedup 1.0000x reference)
import jax
import jax.numpy as jnp
from jax.experimental import pallas as pl


def kernel(edge_index, edge_attr, c1_W1, c1_b1, c1_W2, c1_b2, c1_Wo, c1_bo, c2_W1, c2_b1, c2_W2, c2_b2, c2_Wo, c2_bo):
    raise NotImplementedError("write your pallas kernel here")


# speedup vs baseline: 7.8037x; 7.8037x over previous
import functools

import jax
import jax.numpy as jnp
from jax import lax
from jax.experimental import pallas as pl
from jax.experimental.pallas import tpu as pltpu
from jax.experimental.pallas import tpu_sc as plsc

N = 100000
D = 16
E = 3200000
CHUNK = 128
E_PAD = 3276800
NPAD = N + 352
NTILES = 32
CHUNKS_PER_TILE = E_PAD // (NTILES * CHUNK)
ROWS_PER_TILE = NPAD // 16


def _mlp_body(x_ref, w1_ref, b1_ref, w2_ref, b2_ref, o_ref):
    h = jnp.maximum(
        jnp.dot(x_ref[...], w1_ref[...], preferred_element_type=jnp.float32)
        + b1_ref[...], 0.0)
    o_ref[...] = (
        jnp.dot(h, w2_ref[...], preferred_element_type=jnp.float32)
        + b2_ref[...])


def _mlp(x, w1, b1, w2, b2):
    blk = 10000
    grid = (N // blk,)
    return pl.pallas_call(
        _mlp_body,
        grid=grid,
        in_specs=[
            pl.BlockSpec((blk, D), lambda i: (i, 0)),
            pl.BlockSpec((D, D), lambda i: (0, 0)),
            pl.BlockSpec((1, D), lambda i: (0, 0)),
            pl.BlockSpec((D, D), lambda i: (0, 0)),
            pl.BlockSpec((1, D), lambda i: (0, 0)),
        ],
        out_specs=pl.BlockSpec((blk, D), lambda i: (i, 0)),
        out_shape=jax.ShapeDtypeStruct((N, D), jnp.float32),
    )(x, w1, b1.reshape(1, D), w2, b2.reshape(1, D))


def _seg_body(table_hbm, src_hbm, dst_hbm, z2_hbm, z1_hbm,
              sums_hbm, cnts_hbm,
              src_v, dst_v, rows_v, ones_v, acc_sh, cnt_sh, sem):
    cid = lax.axis_index("c")
    sid = lax.axis_index("s")
    wid = sid * 2 + cid

    r0 = sid * ROWS_PER_TILE
    pltpu.sync_copy(z2_hbm.at[pl.ds(r0, ROWS_PER_TILE)],
                    acc_sh.at[pl.ds(r0, ROWS_PER_TILE)])
    pltpu.sync_copy(z1_hbm.at[pl.ds(r0, ROWS_PER_TILE)],
                    cnt_sh.at[pl.ds(r0, ROWS_PER_TILE)])
    for i in range(CHUNK // 16):
        ones_v[pl.ds(16 * i, 16)] = jnp.ones((16,), jnp.float32)
    plsc.subcore_barrier()

    def body(j, carry):
        ch = wid * CHUNKS_PER_TILE + j
        pltpu.sync_copy(src_hbm.at[ch], src_v)
        pltpu.sync_copy(dst_hbm.at[ch], dst_v)
        pltpu.async_copy(table_hbm.at[src_v.at[0]], rows_v, sem).wait()
        pltpu.sync_copy(rows_v, acc_sh.at[dst_v.at[0]], add=True)
        pltpu.sync_copy(ones_v, cnt_sh.at[dst_v.at[0]], add=True)
        return carry

    lax.fori_loop(0, CHUNKS_PER_TILE, body, 0)
    plsc.subcore_barrier()

    pltpu.sync_copy(acc_sh.at[pl.ds(r0, ROWS_PER_TILE)],
                    sums_hbm.at[cid].at[pl.ds(r0, ROWS_PER_TILE)])
    pltpu.sync_copy(cnt_sh.at[pl.ds(r0, ROWS_PER_TILE)],
                    cnts_hbm.at[cid].at[pl.ds(r0, ROWS_PER_TILE)])


def _segment(table, src3, dst3, z2, z1):
    mesh = plsc.VectorSubcoreMesh(core_axis_name="c", subcore_axis_name="s")
    fn = pl.kernel(
        _seg_body, mesh=mesh,
        out_type=[
            jax.ShapeDtypeStruct((2, NPAD, D), jnp.float32),
            jax.ShapeDtypeStruct((2, NPAD), jnp.float32),
        ],
        scratch_types=[
            pltpu.VMEM((1, CHUNK), jnp.int32),
            pltpu.VMEM((1, CHUNK), jnp.int32),
            pltpu.VMEM((CHUNK, D), jnp.float32),
            pltpu.VMEM((CHUNK,), jnp.float32),
            pltpu.VMEM_SHARED((NPAD, D), jnp.float32),
            pltpu.VMEM_SHARED((NPAD,), jnp.float32),
            pltpu.SemaphoreType.DMA,
        ],
        compiler_params=pltpu.CompilerParams(use_tc_tiling_on_sc=False),
    )
    return fn(table, src3, dst3, z2, z1)


def _fin_body(s0_ref, s1_ref, ea_ref, c0_ref, c1_ref, wo_ref, bo_ref, o_ref):
    cnt = c0_ref[...] + c1_ref[...] + 1.0
    mean = (s0_ref[...] + s1_ref[...] + ea_ref[...]) / cnt
    o_ref[...] = (
        jnp.dot(mean, wo_ref[...], preferred_element_type=jnp.float32)
        + bo_ref[...])


def _finalize(s0, s1, ea2, c0, c1, wo, bo):
    blk = 2000
    grid = (N // blk,)
    return pl.pallas_call(
        _fin_body,
        grid=grid,
        in_specs=[
            pl.BlockSpec((blk, D), lambda i: (i, 0)),
            pl.BlockSpec((blk, D), lambda i: (i, 0)),
            pl.BlockSpec((blk, D), lambda i: (i, 0)),
            pl.BlockSpec((blk, 1), lambda i: (i, 0)),
            pl.BlockSpec((blk, 1), lambda i: (i, 0)),
            pl.BlockSpec((D, D), lambda i: (0, 0)),
            pl.BlockSpec((1, D), lambda i: (0, 0)),
        ],
        out_specs=pl.BlockSpec((blk, D), lambda i: (i, 0)),
        out_shape=jax.ShapeDtypeStruct((N, D), jnp.float32),
    )(s0, s1, ea2, c0, c1, wo, bo.reshape(1, D))


@jax.jit
def kernel(edge_index, edge_attr,
           c1_W1, c1_b1, c1_W2, c1_b2, c1_Wo, c1_bo,
           c2_W1, c2_b1, c2_W2, c2_b2, c2_Wo, c2_bo):
    ea2 = _mlp(edge_attr[:N], c2_W1, c2_b1, c2_W2, c2_b2)

    pad = E_PAD - E
    src = jnp.concatenate([edge_index[0], jnp.zeros((pad,), jnp.int32)])
    dst = jnp.concatenate([edge_index[1], jnp.full((pad,), N, jnp.int32)])
    src3 = src.reshape(E_PAD // CHUNK, 1, CHUNK)
    dst3 = dst.reshape(E_PAD // CHUNK, 1, CHUNK)
    z2 = jnp.zeros((NPAD, D), jnp.float32)
    z1 = jnp.zeros((NPAD,), jnp.float32)

    sums, cnts = _segment(ea2, src3, dst3, z2, z1)

    s0 = sums[0, :N]
    s1 = sums[1, :N]
    c0 = cnts[0, :N].reshape(N, 1)
    c1 = cnts[1, :N].reshape(N, 1)
    return _finalize(s0, s1, ea2, c0, c1, c2_Wo, c2_bo)
